# baseline (speedup 1.0000x reference)
import functools

import jax
import jax.numpy as jnp
from jax import lax
from jax.experimental import pallas as pl
from jax.experimental.pallas import tpu as pltpu
from jax.experimental.pallas import tpu_sc as plsc

B, T, D = 16, 2048, 64
N = B * T
NW = 32
PER_W = N // NW
CH = 256
NCH = PER_W // CH
OR_CH = CH // 2
LANES = 16


def _emb_body(idx_hbm, tok2_hbm, pos2_hbm, out2_hbm,
              idx_v, hidx_v, par_v, buf_v, pos_v, out_v, gsem, psem):
    cax = lax.axis_index("c")
    sax = lax.axis_index("s")
    wid = sax * 2 + cax

    for ch in range(NCH):
        jbase = pl.multiple_of(wid * PER_W + ch * CH, CH)
        rbase = pl.multiple_of(jbase // 2, OR_CH)
        prow = pl.multiple_of(rbase % (T // 2), OR_CH)

        pltpu.sync_copy(idx_hbm.at[pl.ds(jbase, CH)], idx_v)
        pos_cp = pltpu.async_copy(
            pos2_hbm.at[pl.ds(prow, OR_CH)], pos_v, psem)

        for q in range(CH // LANES):
            sl = pl.ds(q * LANES, LANES)
            v = idx_v[sl]
            hidx_v[sl] = (
                lax.shift_left(lax.shift_right_logical(v, 4), 3) | (v & 7))
            par_v[sl] = (lax.shift_right_logical(v, 3) & 1) * D

        g0 = pltpu.async_copy(
            tok2_hbm.at[hidx_v.at[pl.ds(0, 128)]],
            buf_v.at[pl.ds(0, 128), :], gsem)
        g1 = pltpu.async_copy(
            tok2_hbm.at[hidx_v.at[pl.ds(128, 128)]],
            buf_v.at[pl.ds(128, 128), :], gsem)
        g0.wait()
        g1.wait()
        pos_cp.wait()

        iota16 = lax.iota(jnp.int32, LANES)

        def row(r, _):
            j0 = 16 * (r // 8) + (r % 8)
            j1 = j0 + 8
            off0 = plsc.load_gather(
                par_v, [jnp.full((LANES,), j0, jnp.int32)])
            off1 = plsc.load_gather(
                par_v, [jnp.full((LANES,), j1, jnp.int32)])
            row0 = jnp.full((LANES,), j0, jnp.int32)
            row1 = jnp.full((LANES,), j1, jnp.int32)
            for q in range(D // LANES):
                sl = pl.ds(q * LANES, LANES)
                val = plsc.load_gather(
                    buf_v, [row0, off0 + q * LANES + iota16])
                out_v[r, sl] = val + pos_v[r, sl]
            for q in range(D // LANES):
                sl = pl.ds(D + q * LANES, LANES)
                val = plsc.load_gather(
                    buf_v, [row1, off1 + q * LANES + iota16])
                out_v[r, sl] = val + pos_v[r, sl]
            return _

        lax.fori_loop(0, OR_CH, row, 0)

        pltpu.sync_copy(out_v, out2_hbm.at[pl.ds(rbase, OR_CH)])


@jax.jit
def _emb(idx_flat, tok2, pos2):
    mesh = plsc.VectorSubcoreMesh(core_axis_name="c", subcore_axis_name="s")
    return pl.kernel(
        _emb_body,
        out_type=jax.ShapeDtypeStruct((N // 2, 2 * D), jnp.float32),
        mesh=mesh,
        scratch_types=[
            pltpu.VMEM((CH,), jnp.int32),
            pltpu.VMEM((CH,), jnp.int32),
            pltpu.VMEM((CH,), jnp.int32),
            pltpu.VMEM((CH, 2 * D), jnp.float32),
            pltpu.VMEM((OR_CH, 2 * D), jnp.float32),
            pltpu.VMEM((OR_CH, 2 * D), jnp.float32),
            pltpu.SemaphoreType.DMA,
            pltpu.SemaphoreType.DMA,
        ],
        compiler_params=pltpu.CompilerParams(needs_layout_passes=False),
    )(idx_flat, tok2, pos2)


def _pack16(table):
    return (table.reshape(-1, 2, 8, D)
            .transpose(0, 2, 1, 3)
            .reshape(-1, 2 * D))


def kernel(idx, tok_table, pos_table):
    out2 = _emb(idx.reshape(N), _pack16(tok_table), _pack16(pos_table))
    return (out2.reshape(-1, 8, 2, D)
            .transpose(0, 2, 1, 3)
            .reshape(B, T, D))


# speedup vs baseline: 1.0003x; 1.0003x over previous
import functools

import jax
import jax.numpy as jnp
from jax import lax
from jax.experimental import pallas as pl
from jax.experimental.pallas import tpu as pltpu
from jax.experimental.pallas import tpu_sc as plsc

B, T, D = 16, 2048, 64
N = B * T
NW = 32
PER_W = N // NW
CH = 256
NCH = PER_W // CH
OR_CH = CH // 2
LANES = 16


def _emb_body(idx_hbm, tok2_hbm, pos2_hbm, out2_hbm,
              idx_v, hidx_v, par_v, buf_v, pos_v, out_v, gsem, psem):
    cax = lax.axis_index("c")
    sax = lax.axis_index("s")
    wid = sax * 2 + cax

    for ch in range(NCH):
        jbase = pl.multiple_of(wid * PER_W + ch * CH, CH)
        rbase = pl.multiple_of(jbase // 2, OR_CH)
        prow = pl.multiple_of(rbase % (T // 2), OR_CH)

        pltpu.sync_copy(idx_hbm.at[pl.ds(jbase, CH)], idx_v)
        pos_cp = pltpu.async_copy(
            pos2_hbm.at[pl.ds(prow, OR_CH)], pos_v, psem)

        for q in range(CH // LANES):
            sl = pl.ds(q * LANES, LANES)
            v = idx_v[sl]
            hidx_v[sl] = (
                lax.shift_left(lax.shift_right_logical(v, 4), 3) | (v & 7))
            par_v[sl] = (lax.shift_right_logical(v, 3) & 1) * D

        g0 = pltpu.async_copy(
            tok2_hbm.at[hidx_v.at[pl.ds(0, 128)]],
            buf_v.at[pl.ds(0, 128), :], gsem)
        g1 = pltpu.async_copy(
            tok2_hbm.at[hidx_v.at[pl.ds(128, 128)]],
            buf_v.at[pl.ds(128, 128), :], gsem)
        g0.wait()
        g1.wait()
        pos_cp.wait()

        iota16 = lax.iota(jnp.int32, LANES)

        def row(r, _):
            j0 = 16 * (r // 8) + (r % 8)
            j1 = j0 + 8
            off0 = plsc.load_gather(
                par_v, [jnp.full((LANES,), j0, jnp.int32)])
            off1 = plsc.load_gather(
                par_v, [jnp.full((LANES,), j1, jnp.int32)])
            row0 = jnp.full((LANES,), j0, jnp.int32)
            row1 = jnp.full((LANES,), j1, jnp.int32)
            for q in range(D // LANES):
                sl = pl.ds(q * LANES, LANES)
                val = plsc.load_gather(
                    buf_v, [row0, off0 + q * LANES + iota16])
                out_v[r, sl] = val + pos_v[r, sl]
            for q in range(D // LANES):
                sl = pl.ds(D + q * LANES, LANES)
                val = plsc.load_gather(
                    buf_v, [row1, off1 + q * LANES + iota16])
                out_v[r, sl] = val + pos_v[r, sl]
            return _

        lax.fori_loop(0, OR_CH, row, 0)

        pltpu.sync_copy(out_v, out2_hbm.at[pl.ds(rbase, OR_CH)])


@jax.jit
def _emb(idx_flat, tok2, pos2):
    mesh = plsc.VectorSubcoreMesh(core_axis_name="c", subcore_axis_name="s")
    return pl.kernel(
        _emb_body,
        out_type=jax.ShapeDtypeStruct((N // 2, 2 * D), jnp.float32),
        mesh=mesh,
        scratch_types=[
            pltpu.VMEM((CH,), jnp.int32),
            pltpu.VMEM((CH,), jnp.int32),
            pltpu.VMEM((CH,), jnp.int32),
            pltpu.VMEM((CH, 2 * D), jnp.float32),
            pltpu.VMEM((OR_CH, 2 * D), jnp.float32),
            pltpu.VMEM((OR_CH, 2 * D), jnp.float32),
            pltpu.SemaphoreType.DMA,
            pltpu.SemaphoreType.DMA,
        ],
        compiler_params=pltpu.CompilerParams(needs_layout_passes=False),
    )(idx_flat, tok2, pos2)


def _pack16(table):
    return (table.reshape(-1, 2, 8, D)
            .transpose(0, 2, 1, 3)
            .reshape(-1, 2 * D))


def kernel(idx, tok_table, pos_table):
    from jax.experimental.layout import Format, Layout
    from jax.sharding import SingleDeviceSharding
    fmt = Format(Layout(major_to_minor=(0, 1), tiling=((8, 128),)),
                 SingleDeviceSharding(jax.devices()[0]))
    tok_r = jax.device_put(tok_table, fmt)
    out2 = _emb(idx.reshape(N), _pack16(tok_r), _pack16(pos_table))
    return (out2.reshape(-1, 8, 2, D)
            .transpose(0, 2, 1, 3)
            .reshape(B, T, D))


# speedup vs baseline: 2.4635x; 2.4627x over previous
import functools

import jax
import jax.numpy as jnp
from jax import lax
from jax.experimental import pallas as pl
from jax.experimental.pallas import tpu as pltpu
from jax.experimental.pallas import tpu_sc as plsc

B, T, D = 16, 2048, 64
N = B * T
NW = 32
PER_W = N // NW
CHUNK = 128
NCHUNK = PER_W // CHUNK
LANES = 16


def _emb_body(idx_hbm, tok_hbm, pos_hbm, out_hbm,
              idx_v, pos_v, buf0, buf1, gsem0, gsem1, ssem0, ssem1, psem):
    c = lax.axis_index("c")
    s = lax.axis_index("s")
    wid = s * 2 + c
    base = wid * PER_W
    t0 = base % T

    bufs = (buf0, buf1)
    gsems = (gsem0, gsem1)
    ssems = (ssem0, ssem1)

    pltpu.sync_copy(idx_hbm.at[pl.ds(base, PER_W)], idx_v)
    pos_cp = pltpu.async_copy(pos_hbm.at[pl.ds(t0, PER_W)], pos_v, psem)

    gathers = [None] * NCHUNK
    stores = [None] * NCHUNK

    def issue_gather(k):
        b = k % 2
        gathers[k] = pltpu.async_copy(
            tok_hbm.at[idx_v.at[pl.ds(k * CHUNK, CHUNK)]], bufs[b], gsems[b]
        )

    issue_gather(0)
    pos_waited = False

    for k in range(NCHUNK):
        b = k % 2
        gathers[k].wait()
        if k + 1 < NCHUNK:
            if k >= 1:
                stores[k - 1].wait()
            issue_gather(k + 1)
        if not pos_waited:
            pos_cp.wait()
            pos_waited = True

        off = k * CHUNK
        buf = bufs[b]

        @plsc.parallel_loop(0, CHUNK, unroll=8)
        def add_row(r, off=off, buf=buf):
            for q in range(D // LANES):
                sl = pl.ds(q * LANES, LANES)
                buf[r, sl] = buf[r, sl] + pos_v[off + r, sl]

        stores[k] = pltpu.async_copy(
            buf, out_hbm.at[pl.ds(base + off, CHUNK)], ssems[b]
        )

    stores[NCHUNK - 2].wait()
    stores[NCHUNK - 1].wait()


@jax.jit
def _emb(idx_flat, tok_table, pos_table):
    mesh = plsc.VectorSubcoreMesh(core_axis_name="c", subcore_axis_name="s")
    return pl.kernel(
        _emb_body,
        out_type=jax.ShapeDtypeStruct((N, D), jnp.float32),
        mesh=mesh,
        scratch_types=[
            pltpu.VMEM((PER_W,), jnp.int32),
            pltpu.VMEM((PER_W, D), jnp.float32),
            pltpu.VMEM((CHUNK, D), jnp.float32),
            pltpu.VMEM((CHUNK, D), jnp.float32),
            pltpu.SemaphoreType.DMA,
            pltpu.SemaphoreType.DMA,
            pltpu.SemaphoreType.DMA,
            pltpu.SemaphoreType.DMA,
            pltpu.SemaphoreType.DMA,
        ],
        compiler_params=pltpu.CompilerParams(use_tc_tiling_on_sc=False),
    )(idx_flat, tok_table, pos_table)


def kernel(idx, tok_table, pos_table):
    out = _emb(idx.reshape(N), tok_table, pos_table)
    return out.reshape(B, T, D)


# speedup vs baseline: 2.4664x; 1.0012x over previous
import functools

import jax
import jax.numpy as jnp
from jax import lax
from jax.experimental import pallas as pl
from jax.experimental.pallas import tpu as pltpu
from jax.experimental.pallas import tpu_sc as plsc

B, T, D = 16, 2048, 64
N = B * T
NW = 32
PER_W = N // NW
CHUNK = 128
NCHUNK = PER_W // CHUNK
LANES = 16


def _emb_body(idx_hbm, tok_hbm, pos_hbm, out_hbm,
              idx_v, pos_v, buf0, buf1, gsem0, gsem1, ssem0, ssem1, psem):
    c = lax.axis_index("c")
    s = lax.axis_index("s")
    wid = s * 2 + c
    base = wid * PER_W
    t0 = base % T

    bufs = (buf0, buf1)
    gsems = (gsem0, gsem1)
    ssems = (ssem0, ssem1)

    pltpu.sync_copy(idx_hbm.at[pl.ds(base, PER_W)], idx_v)
    pos_cp = pltpu.async_copy(pos_hbm.at[pl.ds(t0, PER_W)], pos_v, psem)

    gathers = [None] * NCHUNK
    stores = [None] * NCHUNK

    def issue_gather(k):
        b = k % 2
        gathers[k] = pltpu.async_copy(
            tok_hbm.at[idx_v.at[pl.ds(k * CHUNK, CHUNK)]], bufs[b], gsems[b]
        )

    issue_gather(0)
    pos_waited = False

    for k in range(NCHUNK):
        b = k % 2
        gathers[k].wait()
        if k + 1 < NCHUNK:
            if k >= 1:
                stores[k - 1].wait()
            issue_gather(k + 1)
        if not pos_waited:
            pos_cp.wait()
            pos_waited = True

        off = k * CHUNK
        buf = bufs[b]

        @plsc.parallel_loop(0, CHUNK, unroll=8)
        def add_row(r, off=off, buf=buf):
            for q in range(D // LANES):
                sl = pl.ds(q * LANES, LANES)
                buf[r, sl] = buf[r, sl] + pos_v[off + r, sl]

        stores[k] = pltpu.async_copy(
            buf, out_hbm.at[pl.ds(base + off, CHUNK)], ssems[b]
        )

    stores[NCHUNK - 2].wait()
    stores[NCHUNK - 1].wait()


@jax.jit
def _emb(idx_flat, tok_table, pos_table):
    mesh = plsc.VectorSubcoreMesh(core_axis_name="c", subcore_axis_name="s")
    return pl.kernel(
        _emb_body,
        out_type=jax.ShapeDtypeStruct((N, D), jnp.float32),
        mesh=mesh,
        scratch_types=[
            pltpu.VMEM((PER_W,), jnp.int32),
            pltpu.VMEM((PER_W, D), jnp.float32),
            pltpu.VMEM((CHUNK, D), jnp.float32),
            pltpu.VMEM((CHUNK, D), jnp.float32),
            pltpu.SemaphoreType.DMA,
            pltpu.SemaphoreType.DMA,
            pltpu.SemaphoreType.DMA,
            pltpu.SemaphoreType.DMA,
            pltpu.SemaphoreType.DMA,
        ],
        compiler_params=pltpu.CompilerParams(
            use_tc_tiling_on_sc=False, skip_device_barrier=True),
    )(idx_flat, tok_table, pos_table)


def kernel(idx, tok_table, pos_table):
    out = _emb(idx.reshape(N), tok_table, pos_table)
    return out.reshape(B, T, D)
